# baseline (speedup 1.0000x reference)
import functools

import jax
import jax.numpy as jnp
from jax import lax
from jax.experimental import pallas as pl
from jax.experimental.pallas import tpu as pltpu
from jax.experimental.pallas import tpu_sc as plsc

B, L, H, V = 4096, 50, 64, 1000000
NC, NS = 2, 16
NW = NC * NS
B_PW = B // NW
GRPS = B_PW // 16
EPS = 1e-5
UNROLL_A = 2
UNROLL_B = 4


def _rsqrt(x):
    i = plsc.bitcast(x, jnp.int32)
    i = 0x5F3759DF - (i >> 1)
    y = plsc.bitcast(i, jnp.float32)
    for _ in range(3):
        y = y * (1.5 - (0.5 * x) * y * y)
    return y


def _ln_chunk(buf, obuf, gsplat, bsplat):
    iota = lax.iota(jnp.int32, 16)
    grp_rows = [iota + 16 * g for g in range(GRPS)]
    zero = jnp.zeros((16,), jnp.float32)

    def pass_a(i, accs):
        out = list(accs)
        for u in range(UNROLL_A):
            h = UNROLL_A * i + u
            col = (h + iota) & (H - 1)
            for g in range(GRPS):
                x = plsc.load_gather(buf, [grp_rows[g], col])
                plsc.store_scatter(obuf, [col, grp_rows[g]], x)
                out[2 * g] = out[2 * g] + x
                out[2 * g + 1] = out[2 * g + 1] + x * x
        return tuple(out)

    accs = lax.fori_loop(0, H // UNROLL_A, pass_a, (zero,) * (2 * GRPS))
    means, rstds = [], []
    for g in range(GRPS):
        mean = accs[2 * g] * (1.0 / H)
        var = accs[2 * g + 1] * (1.0 / H) - mean * mean + EPS
        means.append(mean)
        rstds.append(_rsqrt(var))

    def pass_b(i, carry):
        for u in range(UNROLL_B):
            h = UNROLL_B * i + u
            gv = gsplat[h, :]
            bv = bsplat[h, :]
            for g in range(GRPS):
                x = obuf[h, pl.ds(16 * g, 16)]
                y = (x - means[g]) * rstds[g] * gv + bv
                obuf[h, pl.ds(16 * g, 16)] = y
        return carry

    lax.fori_loop(0, H // UNROLL_B, pass_b, 0)


def _make_embed(with_visit):
    nch = L + 1 if with_visit else L
    out_type = [jax.ShapeDtypeStruct((L, H, B), jnp.float32)]
    if with_visit:
        out_type.append(jax.ShapeDtypeStruct((1, H, B), jnp.float32))

    @functools.partial(
        pl.kernel,
        out_type=tuple(out_type),
        mesh=plsc.VectorSubcoreMesh(core_axis_name="c", subcore_axis_name="s"),
        compiler_params=pltpu.CompilerParams(
            use_tc_tiling_on_sc=False, needs_layout_passes=False),
        scratch_types=[
            pltpu.VMEM((B_PW, L), jnp.int32),
            pltpu.VMEM((nch, B_PW), jnp.int32),
            pltpu.VMEM((B_PW, H), jnp.float32),
            pltpu.VMEM((B_PW, H), jnp.float32),
            pltpu.VMEM((B_PW, H), jnp.float32),
            pltpu.VMEM((B_PW, H), jnp.float32),
            pltpu.VMEM((H, B_PW), jnp.float32),
            pltpu.VMEM((H, B_PW), jnp.float32),
            pltpu.VMEM((H, B_PW), jnp.float32),
            pltpu.VMEM((H, B_PW), jnp.float32),
            pltpu.VMEM((H,), jnp.float32),
            pltpu.VMEM((H,), jnp.float32),
            pltpu.VMEM((H, 16), jnp.float32),
            pltpu.VMEM((H, 16), jnp.float32),
            pltpu.SemaphoreType.DMA,
            pltpu.SemaphoreType.DMA,
            pltpu.SemaphoreType.DMA,
            pltpu.SemaphoreType.DMA,
            pltpu.SemaphoreType.DMA,
            pltpu.SemaphoreType.DMA,
            pltpu.SemaphoreType.DMA,
            pltpu.SemaphoreType.DMA,
        ],
    )
    def _embed(*refs):
        if with_visit:
            (idx_hbm, tab, visit_tab, gamma_b, beta_b, out, visit_out,
             idx_raw, idx_t, buf0, buf1, buf2, buf3,
             obuf0, obuf1, obuf2, obuf3, gamma_v, beta_v, gsplat, bsplat,
             gsem0, gsem1, gsem2, gsem3, fsem0, fsem1, fsem2, fsem3) = refs
        else:
            (idx_hbm, tab, gamma_b, beta_b, out,
             idx_raw, idx_t, buf0, buf1, buf2, buf3,
             obuf0, obuf1, obuf2, obuf3, gamma_v, beta_v, gsplat, bsplat,
             gsem0, gsem1, gsem2, gsem3, fsem0, fsem1, fsem2, fsem3) = refs

        wid = lax.axis_index("s") * NC + lax.axis_index("c")
        b_base = wid * B_PW
        pltpu.sync_copy(idx_hbm.at[pl.ds(b_base, B_PW)], idx_raw)
        pltpu.sync_copy(gamma_b, gamma_v)
        pltpu.sync_copy(beta_b, beta_v)

        iota = lax.iota(jnp.int32, 16)
        zero_i = jnp.zeros((16,), jnp.int32)

        def build_l(l, carry):
            for g in range(GRPS):
                rows = 16 * g + iota
                v = plsc.load_gather(
                    idx_raw, [rows, jnp.full((16,), l, jnp.int32)])
                idx_t[l, pl.ds(16 * g, 16)] = v
            return carry

        lax.fori_loop(0, L, build_l, 0)
        if with_visit:
            for g in range(GRPS):
                idx_t[L, pl.ds(16 * g, 16)] = zero_i

        for q in range(4):
            gq = gamma_v[pl.ds(16 * q, 16)]
            bq = beta_v[pl.ds(16 * q, 16)]
            h_ids = 16 * q + iota
            for c in range(16):
                cols = (c + iota) & 15
                plsc.store_scatter(gsplat, [h_ids, cols], gq)
                plsc.store_scatter(bsplat, [h_ids, cols], bq)

        bufs = (buf0, buf1, buf2, buf3)
        obufs = (obuf0, obuf1, obuf2, obuf3)
        gsems = (gsem0, gsem1, gsem2, gsem3)
        fsems = (fsem0, fsem1, fsem2, fsem3)

        def gather(j, buf, sem):
            idx = idx_t.at[j]
            if with_visit:
                @pl.when(j < L)
                def _():
                    pltpu.async_copy(tab.at[idx], buf, sem)

                @pl.when(j >= L)
                def _():
                    pltpu.async_copy(visit_tab.at[idx], buf, sem)
            else:
                pltpu.async_copy(tab.at[idx], buf, sem)

        def gather_drain(buf, sem):
            pltpu.make_async_copy(tab.at[idx_t.at[0]], buf, sem).wait()

        def flush(j, obuf, sem):
            if with_visit:
                @pl.when(j < L)
                def _():
                    pltpu.async_copy(obuf, out.at[j, :, pl.ds(b_base, B_PW)],
                                     sem)

                @pl.when(j >= L)
                def _():
                    pltpu.async_copy(
                        obuf, visit_out.at[0, :, pl.ds(b_base, B_PW)], sem)
            else:
                pltpu.async_copy(obuf, out.at[j, :, pl.ds(b_base, B_PW)], sem)

        def flush_drain(obuf, sem):
            pltpu.make_async_copy(obuf, out.at[0, :, pl.ds(b_base, B_PW)],
                                  sem).wait()

        for s in range(3):
            gather(s, bufs[s], gsems[s])

        def step(j, s):
            gather_drain(bufs[s], gsems[s])

            @pl.when(j + 3 < nch)
            def _():
                gather(j + 3, bufs[(s + 3) % 4], gsems[(s + 3) % 4])

            @pl.when(j >= 4)
            def _():
                flush_drain(obufs[s], fsems[s])
            _ln_chunk(bufs[s], obufs[s], gsplat, bsplat)
            flush(j, obufs[s], fsems[s])

        def body(k, carry):
            for s in range(4):
                step(4 * k + s, s)
            return carry

        lax.fori_loop(0, nch // 4, body, 0)
        for t in range(nch - nch % 4, nch):
            step(t, t % 4)
        for s in range(4):
            flush_drain(obufs[s], fsems[s])

    return _embed


_embed_dx = _make_embed(with_visit=True)
_embed_proc = _make_embed(with_visit=False)


def kernel(dx_ints1, proc_ints1, number, dx_table, proc_table, visit_table,
           ln_gamma, ln_beta):
    del number
    batch = dx_ints1.shape[0]
    dx_o, visit_o = _embed_dx(dx_ints1, dx_table, visit_table,
                              ln_gamma, ln_beta)
    (proc_o,) = _embed_proc(proc_ints1, proc_table, ln_gamma, ln_beta)
    return (
        jnp.transpose(dx_o, (2, 0, 1)),
        jnp.transpose(proc_o, (2, 0, 1)),
        jnp.transpose(visit_o, (2, 0, 1)),
        jnp.ones((batch, 1), jnp.float32),
    )
